# baseline (speedup 1.0000x reference)
import functools

import jax
import jax.numpy as jnp
from jax import lax
from jax.experimental import pallas as pl
from jax.experimental.pallas import tpu as pltpu
from jax.experimental.pallas import tpu_sc as plsc

N_NODES = 10000
N_EDGES = 320000
D = 128
NUM_GRAPHS = 64
N_CLASSES = 16

NP = 10240
NB = 10
BLK = 1024
EP = 327680
CH = 128
NCHW = EP // (32 * CH)
ROWS_PER_TILE = NP // 16


def _sc_mesh():
    return plsc.VectorSubcoreMesh(core_axis_name="c", subcore_axis_name="s")


def _sc_deg(colp):
    EPW = EP // 32

    @functools.partial(
        pl.kernel,
        out_type=jax.ShapeDtypeStruct((32, NP), jnp.float32),
        mesh=_sc_mesh(),
        scratch_types=[
            pltpu.VMEM((EPW,), jnp.int32),
            pltpu.VMEM((NP,), jnp.float32),
        ],
        compiler_params=pltpu.CompilerParams(needs_layout_passes=False),
    )
    def kdeg(col_hbm, out_hbm, colv, degv):
        cid = lax.axis_index("c")
        sid = lax.axis_index("s")
        wid = sid * 2 + cid
        pltpu.sync_copy(col_hbm.at[pl.ds(wid * EPW, EPW)], colv)

        def zero(j, carry):
            degv[pl.ds(j * 16, 16)] = jnp.zeros((16,), jnp.float32)
            return carry

        lax.fori_loop(0, NP // 16, zero, 0)

        ones = jnp.ones((16,), jnp.float32)

        def scat(i, carry):
            idx = colv[pl.ds(i * 16, 16)]
            plsc.addupdate_scatter(degv, [idx], ones)
            return carry

        lax.fori_loop(0, EPW // 16, scat, 0)
        pltpu.sync_copy(degv, out_hbm.at[wid])

    return kdeg(colp)


def _sc_scatter(y, rowp, colp):
    @functools.partial(
        pl.kernel,
        out_type=jax.ShapeDtypeStruct((2, NP, D), jnp.float32),
        mesh=_sc_mesh(),
        scratch_types=[
            pltpu.VMEM((2, CH), jnp.int32),
            pltpu.VMEM((2, CH), jnp.int32),
            pltpu.VMEM((2, CH, D), jnp.float32),
            pltpu.VMEM_SHARED((NP, D), jnp.float32),
        ]
        + [pltpu.SemaphoreType.DMA] * 2,
        compiler_params=pltpu.CompilerParams(use_tc_tiling_on_sc=False),
    )
    def kscat(y_hbm, row_hbm, col_hbm, out_hbm, rowv, colv, datav, acc, *gsems):
        cid = lax.axis_index("c")
        sid = lax.axis_index("s")
        base = (cid * 16 + sid) * NCHW * CH

        def zrow(j, carry):
            for l in range(D // 16):
                datav[0, j, pl.ds(l * 16, 16)] = jnp.zeros((16,), jnp.float32)
            return carry

        lax.fori_loop(0, CH, zrow, 0)
        for k in range(ROWS_PER_TILE // CH):
            pltpu.sync_copy(
                datav.at[0],
                acc.at[pl.ds(sid * ROWS_PER_TILE + k * CH, CH)],
            )
        plsc.subcore_barrier()

        def load_idx(g, b):
            pltpu.sync_copy(row_hbm.at[pl.ds(base + g * CH, CH)], rowv.at[b])
            pltpu.sync_copy(col_hbm.at[pl.ds(base + g * CH, CH)], colv.at[b])

        def fire_g(b):
            pltpu.async_copy(y_hbm.at[rowv.at[b]], datav.at[b], gsems[b])

        def wait_g(b):
            pltpu.make_async_copy(
                y_hbm.at[rowv.at[b]], datav.at[b], gsems[b]
            ).wait()

        load_idx(0, 0)
        load_idx(1, 1)
        fire_g(0)
        fire_g(1)

        def step(g, b, prefetch):
            wait_g(b)
            pltpu.sync_copy(datav.at[b], acc.at[colv.at[b]], add=True)
            if prefetch:
                load_idx(g + 2, b)
                fire_g(b)

        def outer(o, carry):
            for bi in range(2):
                step(o * 2 + bi, bi, True)
            return carry

        lax.fori_loop(0, (NCHW - 2) // 2, outer, 0)
        step(NCHW - 2, 0, False)
        step(NCHW - 1, 1, False)

        plsc.subcore_barrier()
        pltpu.sync_copy(
            acc.at[pl.ds(sid * ROWS_PER_TILE, ROWS_PER_TILE)],
            out_hbm.at[cid, pl.ds(sid * ROWS_PER_TILE, ROWS_PER_TILE)],
        )

    return kscat(y, rowp, colp)


def _dinv_of(deg_ref):
    dsum = jnp.sum(deg_ref[...], axis=0)
    return lax.rsqrt(1.0 + dsum)[:, None]


def _y1_body(x_ref, w_ref, deg_ref, y_ref):
    dinv = _dinv_of(deg_ref)
    xw = jnp.dot(x_ref[...], w_ref[...], preferred_element_type=jnp.float32)
    y_ref[...] = xw * dinv


def _tc_y1(xP, W1, degparts):
    return pl.pallas_call(
        _y1_body,
        grid=(NB,),
        in_specs=[
            pl.BlockSpec((BLK, D), lambda i: (i, 0)),
            pl.BlockSpec((D, D), lambda i: (0, 0)),
            pl.BlockSpec((32, BLK), lambda i: (0, i)),
        ],
        out_specs=pl.BlockSpec((BLK, D), lambda i: (i, 0)),
        out_shape=jax.ShapeDtypeStruct((NP, D), jnp.float32),
    )(xP, W1, degparts)


def _d1_body(p_ref, y_ref, deg_ref, b_ref, w_ref, out_ref):
    dinv = _dinv_of(deg_ref)
    h = jnp.maximum(
        dinv * (p_ref[0] + p_ref[1] + y_ref[...]) + b_ref[...], 0.0
    )
    out_ref[...] = dinv * jnp.dot(
        h, w_ref[...], preferred_element_type=jnp.float32
    )


def _tc_d1(parts, y1, degparts, b1r, W2):
    return pl.pallas_call(
        _d1_body,
        grid=(NB,),
        in_specs=[
            pl.BlockSpec((2, BLK, D), lambda i: (0, i, 0)),
            pl.BlockSpec((BLK, D), lambda i: (i, 0)),
            pl.BlockSpec((32, BLK), lambda i: (0, i)),
            pl.BlockSpec((1, D), lambda i: (0, 0)),
            pl.BlockSpec((D, D), lambda i: (0, 0)),
        ],
        out_specs=pl.BlockSpec((BLK, D), lambda i: (i, 0)),
        out_shape=jax.ShapeDtypeStruct((NP, D), jnp.float32),
    )(parts, y1, degparts, b1r, W2)


def _d2_body(p_ref, y_ref, deg_ref, b_ref, batch_ref, wc1_ref, bc1_ref,
             wc2_ref, bc2_ref, out_ref, psum, cnt):
    i = pl.program_id(0)

    @pl.when(i == 0)
    def _():
        psum[...] = jnp.zeros_like(psum)
        cnt[...] = jnp.zeros_like(cnt)

    dinv = _dinv_of(deg_ref)
    h = jnp.maximum(
        dinv * (p_ref[0] + p_ref[1] + y_ref[...]) + b_ref[...], 0.0
    )
    b = batch_ref[0, 0, :]
    oh = (b[:, None] == lax.broadcasted_iota(jnp.int32, (BLK, NUM_GRAPHS), 1))
    oh = oh.astype(jnp.float32)
    psum[...] += lax.dot_general(
        oh, h, (((0,), (0,)), ((), ())), preferred_element_type=jnp.float32
    )
    cnt[...] += jnp.broadcast_to(
        jnp.sum(oh, axis=0)[:, None], (NUM_GRAPHS, D)
    )

    @pl.when(i == NB - 1)
    def _():
        pooled = psum[...] / jnp.maximum(cnt[...], 1.0)
        z = jnp.maximum(
            jnp.dot(pooled, wc1_ref[...], preferred_element_type=jnp.float32)
            + bc1_ref[...],
            0.0,
        )
        out_ref[...] = (
            jnp.dot(z, wc2_ref[...], preferred_element_type=jnp.float32)
            + bc2_ref[...]
        )


def _tc_d2(parts, y2, degparts, b2r, batchP, Wc1, bc1r, Wc2, bc2r):
    return pl.pallas_call(
        _d2_body,
        grid=(NB,),
        in_specs=[
            pl.BlockSpec((2, BLK, D), lambda i: (0, i, 0)),
            pl.BlockSpec((BLK, D), lambda i: (i, 0)),
            pl.BlockSpec((32, BLK), lambda i: (0, i)),
            pl.BlockSpec((1, D), lambda i: (0, 0)),
            pl.BlockSpec((1, 1, BLK), lambda i: (i, 0, 0)),
            pl.BlockSpec((D, D), lambda i: (0, 0)),
            pl.BlockSpec((1, D), lambda i: (0, 0)),
            pl.BlockSpec((D, N_CLASSES), lambda i: (0, 0)),
            pl.BlockSpec((1, N_CLASSES), lambda i: (0, 0)),
        ],
        out_specs=pl.BlockSpec((NUM_GRAPHS, N_CLASSES), lambda i: (0, 0)),
        out_shape=jax.ShapeDtypeStruct((NUM_GRAPHS, N_CLASSES), jnp.float32),
        scratch_shapes=[
            pltpu.VMEM((NUM_GRAPHS, D), jnp.float32),
            pltpu.VMEM((NUM_GRAPHS, D), jnp.float32),
        ],
    )(parts, y2, degparts, b2r, batchP, Wc1, bc1r, Wc2, bc2r)


def kernel(x, edge_index, batch, W1, b1, W2, b2, Wc1, bc1, Wc2, bc2):
    ei = edge_index.astype(jnp.int32)
    pad_e = EP - N_EDGES
    rowp = jnp.concatenate([ei[0], jnp.zeros((pad_e,), jnp.int32)])
    colp = jnp.concatenate(
        [ei[1], jnp.full((pad_e,), N_NODES, jnp.int32)]
    )
    xP = jnp.concatenate(
        [x, jnp.zeros((NP - N_NODES, D), jnp.float32)]
    )
    batchP = jnp.concatenate(
        [batch.astype(jnp.int32),
         jnp.full((NP - N_NODES,), NUM_GRAPHS, jnp.int32)]
    ).reshape(NB, 1, BLK)
    b1r = b1.reshape(1, D)
    b2r = b2.reshape(1, D)
    bc1r = bc1.reshape(1, D)
    bc2r = bc2.reshape(1, N_CLASSES)

    degparts = _sc_deg(colp)
    y1 = _tc_y1(xP, W1, degparts)
    p1 = _sc_scatter(y1, rowp, colp)
    y2 = _tc_d1(p1, y1, degparts, b1r, W2)
    p2 = _sc_scatter(y2, rowp, colp)
    return _tc_d2(p2, y2, degparts, b2r, batchP, Wc1, bc1r, Wc2, bc2r)


# speedup vs baseline: 1.0132x; 1.0132x over previous
import functools

import jax
import jax.numpy as jnp
from jax import lax
from jax.experimental import pallas as pl
from jax.experimental.pallas import tpu as pltpu
from jax.experimental.pallas import tpu_sc as plsc

N_NODES = 10000
N_EDGES = 320000
D = 128
NUM_GRAPHS = 64
N_CLASSES = 16

NP = 10240
NB = 10
BLK = 1024
EP = 327680
CH = 64
NCHW = EP // (32 * CH)
ROWS_PER_TILE = NP // 16


def _sc_mesh():
    return plsc.VectorSubcoreMesh(core_axis_name="c", subcore_axis_name="s")


def _sc_deg(colp):
    EPW = EP // 32

    @functools.partial(
        pl.kernel,
        out_type=jax.ShapeDtypeStruct((32, NP), jnp.float32),
        mesh=_sc_mesh(),
        scratch_types=[
            pltpu.VMEM((EPW,), jnp.int32),
            pltpu.VMEM((NP,), jnp.float32),
        ],
        compiler_params=pltpu.CompilerParams(needs_layout_passes=False),
    )
    def kdeg(col_hbm, out_hbm, colv, degv):
        cid = lax.axis_index("c")
        sid = lax.axis_index("s")
        wid = sid * 2 + cid
        pltpu.sync_copy(col_hbm.at[pl.ds(wid * EPW, EPW)], colv)

        def zero(j, carry):
            degv[pl.ds(j * 16, 16)] = jnp.zeros((16,), jnp.float32)
            return carry

        lax.fori_loop(0, NP // 16, zero, 0)

        ones = jnp.ones((16,), jnp.float32)

        def scat(i, carry):
            idx = colv[pl.ds(i * 16, 16)]
            plsc.addupdate_scatter(degv, [idx], ones)
            return carry

        lax.fori_loop(0, EPW // 16, scat, 0)
        pltpu.sync_copy(degv, out_hbm.at[wid])

    return kdeg(colp)


def _sc_scatter(y, rowp2, colp2):
    @functools.partial(
        pl.kernel,
        out_type=jax.ShapeDtypeStruct((2, NP, D), jnp.float32),
        mesh=_sc_mesh(),
        scratch_types=[
            pltpu.VMEM((NCHW, CH), jnp.int32),
            pltpu.VMEM((NCHW, CH), jnp.int32),
            pltpu.VMEM((2, CH, D), jnp.float32),
            pltpu.VMEM_SHARED((NP, D), jnp.float32),
        ]
        + [pltpu.SemaphoreType.DMA] * 2,
        compiler_params=pltpu.CompilerParams(use_tc_tiling_on_sc=False),
    )
    def kscat(y_hbm, row_hbm, col_hbm, out_hbm, rowv, colv, datav, acc, *gsems):
        cid = lax.axis_index("c")
        sid = lax.axis_index("s")

        def zrow(j, carry):
            for l in range(D // 16):
                datav[0, j, pl.ds(l * 16, 16)] = jnp.zeros((16,), jnp.float32)
            return carry

        lax.fori_loop(0, CH, zrow, 0)
        for k in range(ROWS_PER_TILE // CH):
            pltpu.sync_copy(
                datav.at[0],
                acc.at[pl.ds(sid * ROWS_PER_TILE + k * CH, CH)],
            )
        plsc.subcore_barrier()

        w = cid * 16 + sid
        pltpu.sync_copy(row_hbm.at[pl.ds(w * NCHW, NCHW)], rowv)
        pltpu.sync_copy(col_hbm.at[pl.ds(w * NCHW, NCHW)], colv)

        def fire_g(g, b):
            pltpu.async_copy(y_hbm.at[rowv.at[g]], datav.at[b], gsems[b])

        def wait_g(g, b):
            pltpu.make_async_copy(
                y_hbm.at[rowv.at[g]], datav.at[b], gsems[b]
            ).wait()

        fire_g(0, 0)
        fire_g(1, 1)

        def step(g, b, prefetch):
            wait_g(g, b)
            pltpu.sync_copy(datav.at[b], acc.at[colv.at[g]], add=True)
            if prefetch:
                fire_g(g + 2, b)

        def outer(o, carry):
            for bi in range(2):
                step(o * 2 + bi, bi, True)
            return carry

        lax.fori_loop(0, (NCHW - 2) // 2, outer, 0)
        step(NCHW - 2, 0, False)
        step(NCHW - 1, 1, False)

        plsc.subcore_barrier()
        pltpu.sync_copy(
            acc.at[pl.ds(sid * ROWS_PER_TILE, ROWS_PER_TILE)],
            out_hbm.at[cid, pl.ds(sid * ROWS_PER_TILE, ROWS_PER_TILE)],
        )

    return kscat(y, rowp2, colp2)


def _dinv_of(deg_ref):
    dsum = jnp.sum(deg_ref[...], axis=0)
    return lax.rsqrt(1.0 + dsum)[:, None]


def _y1_body(x_ref, w_ref, deg_ref, y_ref):
    dinv = _dinv_of(deg_ref)
    xw = jnp.dot(x_ref[...], w_ref[...], preferred_element_type=jnp.float32)
    y_ref[...] = xw * dinv


def _tc_y1(xP, W1, degparts):
    return pl.pallas_call(
        _y1_body,
        grid=(NB,),
        in_specs=[
            pl.BlockSpec((BLK, D), lambda i: (i, 0)),
            pl.BlockSpec((D, D), lambda i: (0, 0)),
            pl.BlockSpec((32, BLK), lambda i: (0, i)),
        ],
        out_specs=pl.BlockSpec((BLK, D), lambda i: (i, 0)),
        out_shape=jax.ShapeDtypeStruct((NP, D), jnp.float32),
    )(xP, W1, degparts)


def _d1_body(p_ref, y_ref, deg_ref, b_ref, w_ref, out_ref):
    dinv = _dinv_of(deg_ref)
    h = jnp.maximum(
        dinv * (p_ref[0] + p_ref[1] + y_ref[...]) + b_ref[...], 0.0
    )
    out_ref[...] = dinv * jnp.dot(
        h, w_ref[...], preferred_element_type=jnp.float32
    )


def _tc_d1(parts, y1, degparts, b1r, W2):
    return pl.pallas_call(
        _d1_body,
        grid=(NB,),
        in_specs=[
            pl.BlockSpec((2, BLK, D), lambda i: (0, i, 0)),
            pl.BlockSpec((BLK, D), lambda i: (i, 0)),
            pl.BlockSpec((32, BLK), lambda i: (0, i)),
            pl.BlockSpec((1, D), lambda i: (0, 0)),
            pl.BlockSpec((D, D), lambda i: (0, 0)),
        ],
        out_specs=pl.BlockSpec((BLK, D), lambda i: (i, 0)),
        out_shape=jax.ShapeDtypeStruct((NP, D), jnp.float32),
    )(parts, y1, degparts, b1r, W2)


def _d2_body(p_ref, y_ref, deg_ref, b_ref, batch_ref, wc1_ref, bc1_ref,
             wc2_ref, bc2_ref, out_ref, psum, cnt):
    i = pl.program_id(0)

    @pl.when(i == 0)
    def _():
        psum[...] = jnp.zeros_like(psum)
        cnt[...] = jnp.zeros_like(cnt)

    dinv = _dinv_of(deg_ref)
    h = jnp.maximum(
        dinv * (p_ref[0] + p_ref[1] + y_ref[...]) + b_ref[...], 0.0
    )
    b = batch_ref[0, 0, :]
    oh = (b[:, None] == lax.broadcasted_iota(jnp.int32, (BLK, NUM_GRAPHS), 1))
    oh = oh.astype(jnp.float32)
    psum[...] += lax.dot_general(
        oh, h, (((0,), (0,)), ((), ())), preferred_element_type=jnp.float32
    )
    cnt[...] += jnp.broadcast_to(
        jnp.sum(oh, axis=0)[:, None], (NUM_GRAPHS, D)
    )

    @pl.when(i == NB - 1)
    def _():
        pooled = psum[...] / jnp.maximum(cnt[...], 1.0)
        z = jnp.maximum(
            jnp.dot(pooled, wc1_ref[...], preferred_element_type=jnp.float32)
            + bc1_ref[...],
            0.0,
        )
        out_ref[...] = (
            jnp.dot(z, wc2_ref[...], preferred_element_type=jnp.float32)
            + bc2_ref[...]
        )


def _tc_d2(parts, y2, degparts, b2r, batchP, Wc1, bc1r, Wc2, bc2r):
    return pl.pallas_call(
        _d2_body,
        grid=(NB,),
        in_specs=[
            pl.BlockSpec((2, BLK, D), lambda i: (0, i, 0)),
            pl.BlockSpec((BLK, D), lambda i: (i, 0)),
            pl.BlockSpec((32, BLK), lambda i: (0, i)),
            pl.BlockSpec((1, D), lambda i: (0, 0)),
            pl.BlockSpec((1, 1, BLK), lambda i: (i, 0, 0)),
            pl.BlockSpec((D, D), lambda i: (0, 0)),
            pl.BlockSpec((1, D), lambda i: (0, 0)),
            pl.BlockSpec((D, N_CLASSES), lambda i: (0, 0)),
            pl.BlockSpec((1, N_CLASSES), lambda i: (0, 0)),
        ],
        out_specs=pl.BlockSpec((NUM_GRAPHS, N_CLASSES), lambda i: (0, 0)),
        out_shape=jax.ShapeDtypeStruct((NUM_GRAPHS, N_CLASSES), jnp.float32),
        scratch_shapes=[
            pltpu.VMEM((NUM_GRAPHS, D), jnp.float32),
            pltpu.VMEM((NUM_GRAPHS, D), jnp.float32),
        ],
    )(parts, y2, degparts, b2r, batchP, Wc1, bc1r, Wc2, bc2r)


def kernel(x, edge_index, batch, W1, b1, W2, b2, Wc1, bc1, Wc2, bc2):
    ei = edge_index.astype(jnp.int32)
    pad_e = EP - N_EDGES
    rowp = jnp.concatenate([ei[0], jnp.zeros((pad_e,), jnp.int32)])
    colp = jnp.concatenate(
        [ei[1], jnp.full((pad_e,), N_NODES, jnp.int32)]
    )
    xP = jnp.concatenate(
        [x, jnp.zeros((NP - N_NODES, D), jnp.float32)]
    )
    batchP = jnp.concatenate(
        [batch.astype(jnp.int32),
         jnp.full((NP - N_NODES,), NUM_GRAPHS, jnp.int32)]
    ).reshape(NB, 1, BLK)
    b1r = b1.reshape(1, D)
    b2r = b2.reshape(1, D)
    bc1r = bc1.reshape(1, D)
    bc2r = bc2.reshape(1, N_CLASSES)

    rowp2 = rowp.reshape(EP // CH, CH)
    colp2 = colp.reshape(EP // CH, CH)
    degparts = _sc_deg(colp)
    y1 = _tc_y1(xP, W1, degparts)
    p1 = _sc_scatter(y1, rowp2, colp2)
    y2 = _tc_d1(p1, y1, degparts, b1r, W2)
    p2 = _sc_scatter(y2, rowp2, colp2)
    return _tc_d2(p2, y2, degparts, b2r, batchP, Wc1, bc1r, Wc2, bc2r)


# speedup vs baseline: 1.3463x; 1.3288x over previous
import functools

import jax
import jax.numpy as jnp
from jax import lax
from jax.experimental import pallas as pl
from jax.experimental.pallas import tpu as pltpu
from jax.experimental.pallas import tpu_sc as plsc

N_NODES = 10000
N_EDGES = 320000
D = 128
DH = D // 2
NUM_GRAPHS = 64
N_CLASSES = 16

NP = 10240
NB = 10
BLK = 1024
EP = 327680
CH = 128
NCHT = EP // (16 * CH)
RING = 5
ROWS_PER_TILE = NP // 16


def _sc_mesh():
    return plsc.VectorSubcoreMesh(core_axis_name="c", subcore_axis_name="s")


def _sc_deg(colp):
    EPW = EP // 32

    @functools.partial(
        pl.kernel,
        out_type=jax.ShapeDtypeStruct((32, NP), jnp.float32),
        mesh=_sc_mesh(),
        scratch_types=[
            pltpu.VMEM((EPW,), jnp.int32),
            pltpu.VMEM((NP,), jnp.float32),
        ],
        compiler_params=pltpu.CompilerParams(needs_layout_passes=False),
    )
    def kdeg(col_hbm, out_hbm, colv, degv):
        cid = lax.axis_index("c")
        sid = lax.axis_index("s")
        wid = sid * 2 + cid
        pltpu.sync_copy(col_hbm.at[pl.ds(wid * EPW, EPW)], colv)

        def zero(j, carry):
            degv[pl.ds(j * 16, 16)] = jnp.zeros((16,), jnp.float32)
            return carry

        lax.fori_loop(0, NP // 16, zero, 0)

        ones = jnp.ones((16,), jnp.float32)

        def scat(i, carry):
            idx = colv[pl.ds(i * 16, 16)]
            plsc.addupdate_scatter(degv, [idx], ones)
            return carry

        lax.fori_loop(0, EPW // 16, scat, 0)
        pltpu.sync_copy(degv, out_hbm.at[wid])

    return kdeg(colp)


def _sc_scatter(ys, rowp2, colp2):
    @functools.partial(
        pl.kernel,
        out_type=jax.ShapeDtypeStruct((2, NP, DH), jnp.float32),
        mesh=_sc_mesh(),
        scratch_types=[
            pltpu.VMEM((NCHT, CH), jnp.int32),
            pltpu.VMEM((NCHT, CH), jnp.int32),
            pltpu.VMEM((RING, CH, DH), jnp.float32),
            pltpu.VMEM_SHARED((NP, DH), jnp.float32),
        ]
        + [pltpu.SemaphoreType.DMA] * (2 * RING),
        compiler_params=pltpu.CompilerParams(use_tc_tiling_on_sc=False),
    )
    def kscat(y_hbm, row_hbm, col_hbm, out_hbm, rowv, colv, datav, acc, *sems):
        gsems = sems[:RING]
        ssems = sems[RING:]
        cid = lax.axis_index("c")
        sid = lax.axis_index("s")
        ysrc = y_hbm.at[cid]

        def zrow(j, carry):
            for l in range(DH // 16):
                datav[0, j, pl.ds(l * 16, 16)] = jnp.zeros(
                    (16,), jnp.float32
                )
            return carry

        lax.fori_loop(0, CH, zrow, 0)
        for k in range(ROWS_PER_TILE // CH):
            pltpu.sync_copy(
                datav.at[0],
                acc.at[pl.ds(sid * ROWS_PER_TILE + k * CH, CH)],
            )
        plsc.subcore_barrier()

        pltpu.sync_copy(row_hbm.at[pl.ds(sid * NCHT, NCHT)], rowv)
        pltpu.sync_copy(col_hbm.at[pl.ds(sid * NCHT, NCHT)], colv)

        def fire_g(g, b):
            pltpu.async_copy(ysrc.at[rowv.at[g]], datav.at[b], gsems[b])

        def wait_g(g, b):
            pltpu.make_async_copy(
                ysrc.at[rowv.at[g]], datav.at[b], gsems[b]
            ).wait()

        def fire_s(g, b):
            pltpu.async_copy(
                datav.at[b], acc.at[colv.at[g]], ssems[b], add=True
            )

        def wait_s(g, b):
            pltpu.make_async_copy(
                datav.at[b], acc.at[colv.at[g]], ssems[b]
            ).wait()

        fire_g(0, 0)
        fire_g(1, 1)
        for g0 in range(3):
            wait_g(g0, g0)
            fire_s(g0, g0)
            fire_g(g0 + 2, g0 + 2)

        def outer(o, carry):
            for bi in range(RING):
                g = 3 + o * RING + bi
                b = (3 + bi) % RING
                wait_g(g, b)
                fire_s(g, b)
                wait_s(g - 3, bi)
                fire_g(g + 2, bi)
            return carry

        lax.fori_loop(0, (NCHT - 5) // RING, outer, 0)
        for g0 in range(NCHT - 2, NCHT):
            b = g0 % RING
            wait_g(g0, b)
            fire_s(g0, b)
            wait_s(g0 - 3, (g0 + 2) % RING)
        for g0 in range(NCHT - 3, NCHT):
            wait_s(g0, g0 % RING)

        plsc.subcore_barrier()
        pltpu.sync_copy(
            acc.at[pl.ds(sid * ROWS_PER_TILE, ROWS_PER_TILE)],
            out_hbm.at[cid, pl.ds(sid * ROWS_PER_TILE, ROWS_PER_TILE)],
        )

    return kscat(ys, rowp2, colp2)


def _dinv_of(deg_ref):
    dsum = jnp.sum(deg_ref[...], axis=0)
    return lax.rsqrt(1.0 + dsum)[:, None]


def _split(ref):
    return jnp.concatenate([ref[0], ref[1]], axis=1)


def _store_split(ref, val):
    ref[0] = val[:, :DH]
    ref[1] = val[:, DH:]


def _y1_body(x_ref, w_ref, deg_ref, y_ref):
    dinv = _dinv_of(deg_ref)
    xw = jnp.dot(x_ref[...], w_ref[...], preferred_element_type=jnp.float32)
    _store_split(y_ref, xw * dinv)


def _tc_y1(xP, W1, degparts):
    return pl.pallas_call(
        _y1_body,
        grid=(NB,),
        in_specs=[
            pl.BlockSpec((BLK, D), lambda i: (i, 0)),
            pl.BlockSpec((D, D), lambda i: (0, 0)),
            pl.BlockSpec((32, BLK), lambda i: (0, i)),
        ],
        out_specs=pl.BlockSpec((2, BLK, DH), lambda i: (0, i, 0)),
        out_shape=jax.ShapeDtypeStruct((2, NP, DH), jnp.float32),
    )(xP, W1, degparts)


def _d1_body(p_ref, y_ref, deg_ref, b_ref, w_ref, out_ref):
    dinv = _dinv_of(deg_ref)
    h = jnp.maximum(
        dinv * (_split(p_ref) + _split(y_ref)) + b_ref[...], 0.0
    )
    xw = jnp.dot(h, w_ref[...], preferred_element_type=jnp.float32)
    _store_split(out_ref, xw * dinv)


def _tc_d1(parts, y1, degparts, b1r, W2):
    return pl.pallas_call(
        _d1_body,
        grid=(NB,),
        in_specs=[
            pl.BlockSpec((2, BLK, DH), lambda i: (0, i, 0)),
            pl.BlockSpec((2, BLK, DH), lambda i: (0, i, 0)),
            pl.BlockSpec((32, BLK), lambda i: (0, i)),
            pl.BlockSpec((1, D), lambda i: (0, 0)),
            pl.BlockSpec((D, D), lambda i: (0, 0)),
        ],
        out_specs=pl.BlockSpec((2, BLK, DH), lambda i: (0, i, 0)),
        out_shape=jax.ShapeDtypeStruct((2, NP, DH), jnp.float32),
    )(parts, y1, degparts, b1r, W2)


def _d2_body(p_ref, y_ref, deg_ref, b_ref, batch_ref, wc1_ref, bc1_ref,
             wc2_ref, bc2_ref, out_ref, psum, cnt):
    i = pl.program_id(0)

    @pl.when(i == 0)
    def _():
        psum[...] = jnp.zeros_like(psum)
        cnt[...] = jnp.zeros_like(cnt)

    dinv = _dinv_of(deg_ref)
    h = jnp.maximum(
        dinv * (_split(p_ref) + _split(y_ref)) + b_ref[...], 0.0
    )
    b = batch_ref[0, 0, :]
    oh = (b[:, None] == lax.broadcasted_iota(jnp.int32, (BLK, NUM_GRAPHS), 1))
    oh = oh.astype(jnp.float32)
    psum[...] += lax.dot_general(
        oh, h, (((0,), (0,)), ((), ())), preferred_element_type=jnp.float32
    )
    cnt[...] += jnp.broadcast_to(
        jnp.sum(oh, axis=0)[:, None], (NUM_GRAPHS, D)
    )

    @pl.when(i == NB - 1)
    def _():
        pooled = psum[...] / jnp.maximum(cnt[...], 1.0)
        z = jnp.maximum(
            jnp.dot(pooled, wc1_ref[...], preferred_element_type=jnp.float32)
            + bc1_ref[...],
            0.0,
        )
        out_ref[...] = (
            jnp.dot(z, wc2_ref[...], preferred_element_type=jnp.float32)
            + bc2_ref[...]
        )


def _tc_d2(parts, y2, degparts, b2r, batchP, Wc1, bc1r, Wc2, bc2r):
    return pl.pallas_call(
        _d2_body,
        grid=(NB,),
        in_specs=[
            pl.BlockSpec((2, BLK, DH), lambda i: (0, i, 0)),
            pl.BlockSpec((2, BLK, DH), lambda i: (0, i, 0)),
            pl.BlockSpec((32, BLK), lambda i: (0, i)),
            pl.BlockSpec((1, D), lambda i: (0, 0)),
            pl.BlockSpec((1, 1, BLK), lambda i: (i, 0, 0)),
            pl.BlockSpec((D, D), lambda i: (0, 0)),
            pl.BlockSpec((1, D), lambda i: (0, 0)),
            pl.BlockSpec((D, N_CLASSES), lambda i: (0, 0)),
            pl.BlockSpec((1, N_CLASSES), lambda i: (0, 0)),
        ],
        out_specs=pl.BlockSpec((NUM_GRAPHS, N_CLASSES), lambda i: (0, 0)),
        out_shape=jax.ShapeDtypeStruct((NUM_GRAPHS, N_CLASSES), jnp.float32),
        scratch_shapes=[
            pltpu.VMEM((NUM_GRAPHS, D), jnp.float32),
            pltpu.VMEM((NUM_GRAPHS, D), jnp.float32),
        ],
    )(parts, y2, degparts, b2r, batchP, Wc1, bc1r, Wc2, bc2r)


def kernel(x, edge_index, batch, W1, b1, W2, b2, Wc1, bc1, Wc2, bc2):
    ei = edge_index.astype(jnp.int32)
    pad_e = EP - N_EDGES
    rowp = jnp.concatenate([ei[0], jnp.zeros((pad_e,), jnp.int32)])
    colp = jnp.concatenate(
        [ei[1], jnp.full((pad_e,), N_NODES, jnp.int32)]
    )
    rowp2 = rowp.reshape(EP // CH, CH)
    colp2 = colp.reshape(EP // CH, CH)
    xP = jnp.concatenate(
        [x, jnp.zeros((NP - N_NODES, D), jnp.float32)]
    )
    batchP = jnp.concatenate(
        [batch.astype(jnp.int32),
         jnp.full((NP - N_NODES,), NUM_GRAPHS, jnp.int32)]
    ).reshape(NB, 1, BLK)
    b1r = b1.reshape(1, D)
    b2r = b2.reshape(1, D)
    bc1r = bc1.reshape(1, D)
    bc2r = bc2.reshape(1, N_CLASSES)

    degparts = _sc_deg(colp)
    y1 = _tc_y1(xP, W1, degparts)
    p1 = _sc_scatter(y1, rowp2, colp2)
    y2 = _tc_d1(p1, y1, degparts, b1r, W2)
    p2 = _sc_scatter(y2, rowp2, colp2)
    return _tc_d2(p2, y2, degparts, b2r, batchP, Wc1, bc1r, Wc2, bc2r)


# speedup vs baseline: 2.3577x; 1.7513x over previous
import functools

import jax
import jax.numpy as jnp
from jax import lax
from jax.experimental import pallas as pl
from jax.experimental.pallas import tpu as pltpu
from jax.experimental.pallas import tpu_sc as plsc

N_NODES = 10000
N_EDGES = 320000
D = 128
DH = D // 2
NUM_GRAPHS = 64
N_CLASSES = 16

NP = 10240
NB = 10
BLK = 1024
EP = 327680
CH = 128
NCHT = EP // (16 * CH)
NH = 2
NCHH = NCHT // NH
ROWS_PER_TILE = NP // 16


def _sc_mesh():
    return plsc.VectorSubcoreMesh(core_axis_name="c", subcore_axis_name="s")


def _sc_deg(colp):
    EPW = EP // 32

    @functools.partial(
        pl.kernel,
        out_type=jax.ShapeDtypeStruct((32, NP), jnp.float32),
        mesh=_sc_mesh(),
        scratch_types=[
            pltpu.VMEM((EPW,), jnp.int32),
            pltpu.VMEM((NP,), jnp.float32),
        ],
        compiler_params=pltpu.CompilerParams(needs_layout_passes=False),
    )
    def kdeg(col_hbm, out_hbm, colv, degv):
        cid = lax.axis_index("c")
        sid = lax.axis_index("s")
        wid = sid * 2 + cid
        pltpu.sync_copy(col_hbm.at[pl.ds(wid * EPW, EPW)], colv)

        def zero(j, carry):
            degv[pl.ds(j * 16, 16)] = jnp.zeros((16,), jnp.float32)
            return carry

        lax.fori_loop(0, NP // 16, zero, 0)

        ones = jnp.ones((16,), jnp.float32)

        def scat(i, carry):
            idx = colv[pl.ds(i * 16, 16)]
            plsc.addupdate_scatter(degv, [idx], ones)
            return carry

        lax.fori_loop(0, EPW // 16, scat, 0)
        pltpu.sync_copy(degv, out_hbm.at[wid])

    return kdeg(colp)


def _sc_scatter(ys, rowp2, colp2):
    @functools.partial(
        pl.kernel,
        out_type=jax.ShapeDtypeStruct((2, NP, DH), jnp.float32),
        mesh=_sc_mesh(),
        scratch_types=[
            pltpu.VMEM((NCHH, CH), jnp.int32),
            pltpu.VMEM((NCHH, CH), jnp.int32),
            pltpu.VMEM((2, CH, DH), jnp.float32),
            pltpu.VMEM_SHARED((NP, DH), jnp.float32),
            pltpu.VMEM_SHARED((NP, DH), jnp.float32),
        ]
        + [pltpu.SemaphoreType.DMA] * 2,
        compiler_params=pltpu.CompilerParams(use_tc_tiling_on_sc=False),
    )
    def kscat(y_hbm, row_hbm, col_hbm, out_hbm, rowv, colv, datav, acc,
              ymem, *gsems):
        cid = lax.axis_index("c")
        sid = lax.axis_index("s")
        ysrc = y_hbm.at[cid]

        def zrow(j, carry):
            for l in range(DH // 16):
                datav[0, j, pl.ds(l * 16, 16)] = jnp.zeros(
                    (16,), jnp.float32
                )
            return carry

        lax.fori_loop(0, CH, zrow, 0)
        for k in range(ROWS_PER_TILE // CH):
            pltpu.sync_copy(
                datav.at[0],
                acc.at[pl.ds(sid * ROWS_PER_TILE + k * CH, CH)],
            )
        pltpu.sync_copy(
            ysrc.at[pl.ds(sid * ROWS_PER_TILE, ROWS_PER_TILE)],
            ymem.at[pl.ds(sid * ROWS_PER_TILE, ROWS_PER_TILE)],
        )
        plsc.subcore_barrier()

        def fire_g(g, b):
            pltpu.async_copy(ymem.at[rowv.at[g]], datav.at[b], gsems[b])

        def wait_g(g, b):
            pltpu.make_async_copy(
                ymem.at[rowv.at[g]], datav.at[b], gsems[b]
            ).wait()

        for h in range(NH):
            pltpu.sync_copy(
                row_hbm.at[pl.ds(sid * NCHT + h * NCHH, NCHH)], rowv
            )
            pltpu.sync_copy(
                col_hbm.at[pl.ds(sid * NCHT + h * NCHH, NCHH)], colv
            )
            fire_g(0, 0)
            fire_g(1, 1)

            def step(g, b, prefetch):
                wait_g(g, b)
                pltpu.sync_copy(
                    datav.at[b], acc.at[colv.at[g]], add=True
                )
                if prefetch:
                    fire_g(g + 2, b)

            def outer(o, carry):
                for bi in range(2):
                    step(o * 2 + bi, bi, True)
                return carry

            lax.fori_loop(0, (NCHH - 2) // 2, outer, 0)
            step(NCHH - 2, 0, False)
            step(NCHH - 1, 1, False)

        plsc.subcore_barrier()
        pltpu.sync_copy(
            acc.at[pl.ds(sid * ROWS_PER_TILE, ROWS_PER_TILE)],
            out_hbm.at[cid, pl.ds(sid * ROWS_PER_TILE, ROWS_PER_TILE)],
        )

    return kscat(ys, rowp2, colp2)


def _dinv_of(deg_ref):
    dsum = jnp.sum(deg_ref[...], axis=0)
    return lax.rsqrt(1.0 + dsum)[:, None]


def _split(ref):
    return jnp.concatenate([ref[0], ref[1]], axis=1)


def _store_split(ref, val):
    ref[0] = val[:, :DH]
    ref[1] = val[:, DH:]


def _y1_body(x_ref, w_ref, deg_ref, y_ref):
    dinv = _dinv_of(deg_ref)
    xw = jnp.dot(x_ref[...], w_ref[...], preferred_element_type=jnp.float32)
    _store_split(y_ref, xw * dinv)


def _tc_y1(xP, W1, degparts):
    return pl.pallas_call(
        _y1_body,
        grid=(NB,),
        in_specs=[
            pl.BlockSpec((BLK, D), lambda i: (i, 0)),
            pl.BlockSpec((D, D), lambda i: (0, 0)),
            pl.BlockSpec((32, BLK), lambda i: (0, i)),
        ],
        out_specs=pl.BlockSpec((2, BLK, DH), lambda i: (0, i, 0)),
        out_shape=jax.ShapeDtypeStruct((2, NP, DH), jnp.float32),
    )(xP, W1, degparts)


def _d1_body(p_ref, y_ref, deg_ref, b_ref, w_ref, out_ref):
    dinv = _dinv_of(deg_ref)
    h = jnp.maximum(
        dinv * (_split(p_ref) + _split(y_ref)) + b_ref[...], 0.0
    )
    xw = jnp.dot(h, w_ref[...], preferred_element_type=jnp.float32)
    _store_split(out_ref, xw * dinv)


def _tc_d1(parts, y1, degparts, b1r, W2):
    return pl.pallas_call(
        _d1_body,
        grid=(NB,),
        in_specs=[
            pl.BlockSpec((2, BLK, DH), lambda i: (0, i, 0)),
            pl.BlockSpec((2, BLK, DH), lambda i: (0, i, 0)),
            pl.BlockSpec((32, BLK), lambda i: (0, i)),
            pl.BlockSpec((1, D), lambda i: (0, 0)),
            pl.BlockSpec((D, D), lambda i: (0, 0)),
        ],
        out_specs=pl.BlockSpec((2, BLK, DH), lambda i: (0, i, 0)),
        out_shape=jax.ShapeDtypeStruct((2, NP, DH), jnp.float32),
    )(parts, y1, degparts, b1r, W2)


def _d2_body(p_ref, y_ref, deg_ref, b_ref, batch_ref, wc1_ref, bc1_ref,
             wc2_ref, bc2_ref, out_ref, psum, cnt):
    i = pl.program_id(0)

    @pl.when(i == 0)
    def _():
        psum[...] = jnp.zeros_like(psum)
        cnt[...] = jnp.zeros_like(cnt)

    dinv = _dinv_of(deg_ref)
    h = jnp.maximum(
        dinv * (_split(p_ref) + _split(y_ref)) + b_ref[...], 0.0
    )
    b = batch_ref[0, 0, :]
    oh = (b[:, None] == lax.broadcasted_iota(jnp.int32, (BLK, NUM_GRAPHS), 1))
    oh = oh.astype(jnp.float32)
    psum[...] += lax.dot_general(
        oh, h, (((0,), (0,)), ((), ())), preferred_element_type=jnp.float32
    )
    cnt[...] += jnp.broadcast_to(
        jnp.sum(oh, axis=0)[:, None], (NUM_GRAPHS, D)
    )

    @pl.when(i == NB - 1)
    def _():
        pooled = psum[...] / jnp.maximum(cnt[...], 1.0)
        z = jnp.maximum(
            jnp.dot(pooled, wc1_ref[...], preferred_element_type=jnp.float32)
            + bc1_ref[...],
            0.0,
        )
        out_ref[...] = (
            jnp.dot(z, wc2_ref[...], preferred_element_type=jnp.float32)
            + bc2_ref[...]
        )


def _tc_d2(parts, y2, degparts, b2r, batchP, Wc1, bc1r, Wc2, bc2r):
    return pl.pallas_call(
        _d2_body,
        grid=(NB,),
        in_specs=[
            pl.BlockSpec((2, BLK, DH), lambda i: (0, i, 0)),
            pl.BlockSpec((2, BLK, DH), lambda i: (0, i, 0)),
            pl.BlockSpec((32, BLK), lambda i: (0, i)),
            pl.BlockSpec((1, D), lambda i: (0, 0)),
            pl.BlockSpec((1, 1, BLK), lambda i: (i, 0, 0)),
            pl.BlockSpec((D, D), lambda i: (0, 0)),
            pl.BlockSpec((1, D), lambda i: (0, 0)),
            pl.BlockSpec((D, N_CLASSES), lambda i: (0, 0)),
            pl.BlockSpec((1, N_CLASSES), lambda i: (0, 0)),
        ],
        out_specs=pl.BlockSpec((NUM_GRAPHS, N_CLASSES), lambda i: (0, 0)),
        out_shape=jax.ShapeDtypeStruct((NUM_GRAPHS, N_CLASSES), jnp.float32),
        scratch_shapes=[
            pltpu.VMEM((NUM_GRAPHS, D), jnp.float32),
            pltpu.VMEM((NUM_GRAPHS, D), jnp.float32),
        ],
    )(parts, y2, degparts, b2r, batchP, Wc1, bc1r, Wc2, bc2r)


def kernel(x, edge_index, batch, W1, b1, W2, b2, Wc1, bc1, Wc2, bc2):
    ei = edge_index.astype(jnp.int32)
    pad_e = EP - N_EDGES
    rowp = jnp.concatenate([ei[0], jnp.zeros((pad_e,), jnp.int32)])
    colp = jnp.concatenate(
        [ei[1], jnp.full((pad_e,), N_NODES, jnp.int32)]
    )
    rowp2 = rowp.reshape(EP // CH, CH)
    colp2 = colp.reshape(EP // CH, CH)
    xP = jnp.concatenate(
        [x, jnp.zeros((NP - N_NODES, D), jnp.float32)]
    )
    batchP = jnp.concatenate(
        [batch.astype(jnp.int32),
         jnp.full((NP - N_NODES,), NUM_GRAPHS, jnp.int32)]
    ).reshape(NB, 1, BLK)
    b1r = b1.reshape(1, D)
    b2r = b2.reshape(1, D)
    bc1r = bc1.reshape(1, D)
    bc2r = bc2.reshape(1, N_CLASSES)

    degparts = _sc_deg(colp)
    y1 = _tc_y1(xP, W1, degparts)
    p1 = _sc_scatter(y1, rowp2, colp2)
    y2 = _tc_d1(p1, y1, degparts, b1r, W2)
    p2 = _sc_scatter(y2, rowp2, colp2)
    return _tc_d2(p2, y2, degparts, b2r, batchP, Wc1, bc1r, Wc2, bc2r)


# speedup vs baseline: 2.6753x; 1.1347x over previous
import functools

import jax
import jax.numpy as jnp
from jax import lax
from jax.experimental import pallas as pl
from jax.experimental.pallas import tpu as pltpu
from jax.experimental.pallas import tpu_sc as plsc

N_NODES = 10000
N_EDGES = 320000
D = 128
DH = D // 2
NUM_GRAPHS = 64
N_CLASSES = 16

NP = 10240
NB = 10
BLK = 1024
EP = 327680
CH = 128
NCHT = EP // (16 * CH)
NH = 4
NCHH = NCHT // NH
RING = 4
ROWS_PER_TILE = NP // 16


def _sc_mesh():
    return plsc.VectorSubcoreMesh(core_axis_name="c", subcore_axis_name="s")


def _sc_deg(colp):
    EPW = EP // 32

    @functools.partial(
        pl.kernel,
        out_type=jax.ShapeDtypeStruct((32, NP), jnp.float32),
        mesh=_sc_mesh(),
        scratch_types=[
            pltpu.VMEM((EPW,), jnp.int32),
            pltpu.VMEM((NP,), jnp.float32),
        ],
        compiler_params=pltpu.CompilerParams(needs_layout_passes=False),
    )
    def kdeg(col_hbm, out_hbm, colv, degv):
        cid = lax.axis_index("c")
        sid = lax.axis_index("s")
        wid = sid * 2 + cid
        pltpu.sync_copy(col_hbm.at[pl.ds(wid * EPW, EPW)], colv)

        def zero(j, carry):
            degv[pl.ds(j * 16, 16)] = jnp.zeros((16,), jnp.float32)
            return carry

        lax.fori_loop(0, NP // 16, zero, 0)

        ones = jnp.ones((16,), jnp.float32)

        def scat(i, carry):
            idx = colv[pl.ds(i * 16, 16)]
            plsc.addupdate_scatter(degv, [idx], ones)
            return carry

        lax.fori_loop(0, EPW // 16, scat, 0)
        pltpu.sync_copy(degv, out_hbm.at[wid])

    return kdeg(colp)


def _sc_scatter(ys, rowp2, colp2):
    @functools.partial(
        pl.kernel,
        out_type=jax.ShapeDtypeStruct((2, NP, DH), jnp.float32),
        mesh=_sc_mesh(),
        scratch_types=[
            pltpu.VMEM((NCHH, CH), jnp.int32),
            pltpu.VMEM((NCHH, CH), jnp.int32),
            pltpu.VMEM((RING, CH, DH), jnp.float32),
            pltpu.VMEM_SHARED((NP, DH), jnp.float32),
            pltpu.VMEM_SHARED((NP, DH), jnp.float32),
        ]
        + [pltpu.SemaphoreType.DMA] * (2 * RING),
        compiler_params=pltpu.CompilerParams(use_tc_tiling_on_sc=False),
    )
    def kscat(y_hbm, row_hbm, col_hbm, out_hbm, rowv, colv, datav, acc,
              ymem, *sems):
        gsems = sems[:RING]
        ssems = sems[RING:]
        cid = lax.axis_index("c")
        sid = lax.axis_index("s")
        ysrc = y_hbm.at[cid]

        def zrow(j, carry):
            for l in range(DH // 16):
                datav[0, j, pl.ds(l * 16, 16)] = jnp.zeros(
                    (16,), jnp.float32
                )
            return carry

        lax.fori_loop(0, CH, zrow, 0)
        for k in range(ROWS_PER_TILE // CH):
            pltpu.sync_copy(
                datav.at[0],
                acc.at[pl.ds(sid * ROWS_PER_TILE + k * CH, CH)],
            )
        pltpu.sync_copy(
            ysrc.at[pl.ds(sid * ROWS_PER_TILE, ROWS_PER_TILE)],
            ymem.at[pl.ds(sid * ROWS_PER_TILE, ROWS_PER_TILE)],
        )
        plsc.subcore_barrier()

        def fire_g(g, b):
            pltpu.async_copy(ymem.at[rowv.at[g]], datav.at[b], gsems[b])

        def wait_g(g, b):
            pltpu.make_async_copy(
                ymem.at[rowv.at[g]], datav.at[b], gsems[b]
            ).wait()

        def fire_s(g, b):
            pltpu.async_copy(
                datav.at[b], acc.at[colv.at[g]], ssems[b], add=True
            )

        def wait_s(g, b):
            pltpu.make_async_copy(
                datav.at[b], acc.at[colv.at[g]], ssems[b]
            ).wait()

        for h in range(NH):
            pltpu.sync_copy(
                row_hbm.at[pl.ds(sid * NCHT + h * NCHH, NCHH)], rowv
            )
            pltpu.sync_copy(
                col_hbm.at[pl.ds(sid * NCHT + h * NCHH, NCHH)], colv
            )
            fire_g(0, 0)
            fire_g(1, 1)
            for g0 in range(2):
                wait_g(g0, g0)
                fire_s(g0, g0)
                fire_g(g0 + 2, g0 + 2)

            def outer(o, carry):
                for bi in range(RING):
                    g = 2 + o * RING + bi
                    b = (2 + bi) % RING
                    wait_g(g, b)
                    fire_s(g, b)
                    wait_s(g - 2, bi)
                    fire_g(g + 2, bi)
                return carry

            lax.fori_loop(0, (NCHH - 4) // RING, outer, 0)
            for g0 in range(NCHH - 2, NCHH):
                b = g0 % RING
                wait_g(g0, b)
                fire_s(g0, b)
                wait_s(g0 - 2, (g0 + 2) % RING)
            for g0 in range(NCHH - 2, NCHH):
                wait_s(g0, g0 % RING)

        plsc.subcore_barrier()
        pltpu.sync_copy(
            acc.at[pl.ds(sid * ROWS_PER_TILE, ROWS_PER_TILE)],
            out_hbm.at[cid, pl.ds(sid * ROWS_PER_TILE, ROWS_PER_TILE)],
        )

    return kscat(ys, rowp2, colp2)


def _dinv_of(deg_ref):
    dsum = jnp.sum(deg_ref[...], axis=0)
    return lax.rsqrt(1.0 + dsum)[:, None]


def _split(ref):
    return jnp.concatenate([ref[0], ref[1]], axis=1)


def _store_split(ref, val):
    ref[0] = val[:, :DH]
    ref[1] = val[:, DH:]


def _y1_body(x_ref, w_ref, deg_ref, y_ref):
    dinv = _dinv_of(deg_ref)
    xw = jnp.dot(x_ref[...], w_ref[...], preferred_element_type=jnp.float32)
    _store_split(y_ref, xw * dinv)


def _tc_y1(xP, W1, degparts):
    return pl.pallas_call(
        _y1_body,
        grid=(NB,),
        in_specs=[
            pl.BlockSpec((BLK, D), lambda i: (i, 0)),
            pl.BlockSpec((D, D), lambda i: (0, 0)),
            pl.BlockSpec((32, BLK), lambda i: (0, i)),
        ],
        out_specs=pl.BlockSpec((2, BLK, DH), lambda i: (0, i, 0)),
        out_shape=jax.ShapeDtypeStruct((2, NP, DH), jnp.float32),
    )(xP, W1, degparts)


def _d1_body(p_ref, y_ref, deg_ref, b_ref, w_ref, out_ref):
    dinv = _dinv_of(deg_ref)
    h = jnp.maximum(
        dinv * (_split(p_ref) + _split(y_ref)) + b_ref[...], 0.0
    )
    xw = jnp.dot(h, w_ref[...], preferred_element_type=jnp.float32)
    _store_split(out_ref, xw * dinv)


def _tc_d1(parts, y1, degparts, b1r, W2):
    return pl.pallas_call(
        _d1_body,
        grid=(NB,),
        in_specs=[
            pl.BlockSpec((2, BLK, DH), lambda i: (0, i, 0)),
            pl.BlockSpec((2, BLK, DH), lambda i: (0, i, 0)),
            pl.BlockSpec((32, BLK), lambda i: (0, i)),
            pl.BlockSpec((1, D), lambda i: (0, 0)),
            pl.BlockSpec((D, D), lambda i: (0, 0)),
        ],
        out_specs=pl.BlockSpec((2, BLK, DH), lambda i: (0, i, 0)),
        out_shape=jax.ShapeDtypeStruct((2, NP, DH), jnp.float32),
    )(parts, y1, degparts, b1r, W2)


def _d2_body(p_ref, y_ref, deg_ref, b_ref, batch_ref, wc1_ref, bc1_ref,
             wc2_ref, bc2_ref, out_ref, psum, cnt):
    i = pl.program_id(0)

    @pl.when(i == 0)
    def _():
        psum[...] = jnp.zeros_like(psum)
        cnt[...] = jnp.zeros_like(cnt)

    dinv = _dinv_of(deg_ref)
    h = jnp.maximum(
        dinv * (_split(p_ref) + _split(y_ref)) + b_ref[...], 0.0
    )
    b = batch_ref[0, 0, :]
    oh = (b[:, None] == lax.broadcasted_iota(jnp.int32, (BLK, NUM_GRAPHS), 1))
    oh = oh.astype(jnp.float32)
    psum[...] += lax.dot_general(
        oh, h, (((0,), (0,)), ((), ())), preferred_element_type=jnp.float32
    )
    cnt[...] += jnp.broadcast_to(
        jnp.sum(oh, axis=0)[:, None], (NUM_GRAPHS, D)
    )

    @pl.when(i == NB - 1)
    def _():
        pooled = psum[...] / jnp.maximum(cnt[...], 1.0)
        z = jnp.maximum(
            jnp.dot(pooled, wc1_ref[...], preferred_element_type=jnp.float32)
            + bc1_ref[...],
            0.0,
        )
        out_ref[...] = (
            jnp.dot(z, wc2_ref[...], preferred_element_type=jnp.float32)
            + bc2_ref[...]
        )


def _tc_d2(parts, y2, degparts, b2r, batchP, Wc1, bc1r, Wc2, bc2r):
    return pl.pallas_call(
        _d2_body,
        grid=(NB,),
        in_specs=[
            pl.BlockSpec((2, BLK, DH), lambda i: (0, i, 0)),
            pl.BlockSpec((2, BLK, DH), lambda i: (0, i, 0)),
            pl.BlockSpec((32, BLK), lambda i: (0, i)),
            pl.BlockSpec((1, D), lambda i: (0, 0)),
            pl.BlockSpec((1, 1, BLK), lambda i: (i, 0, 0)),
            pl.BlockSpec((D, D), lambda i: (0, 0)),
            pl.BlockSpec((1, D), lambda i: (0, 0)),
            pl.BlockSpec((D, N_CLASSES), lambda i: (0, 0)),
            pl.BlockSpec((1, N_CLASSES), lambda i: (0, 0)),
        ],
        out_specs=pl.BlockSpec((NUM_GRAPHS, N_CLASSES), lambda i: (0, 0)),
        out_shape=jax.ShapeDtypeStruct((NUM_GRAPHS, N_CLASSES), jnp.float32),
        scratch_shapes=[
            pltpu.VMEM((NUM_GRAPHS, D), jnp.float32),
            pltpu.VMEM((NUM_GRAPHS, D), jnp.float32),
        ],
    )(parts, y2, degparts, b2r, batchP, Wc1, bc1r, Wc2, bc2r)


def kernel(x, edge_index, batch, W1, b1, W2, b2, Wc1, bc1, Wc2, bc2):
    ei = edge_index.astype(jnp.int32)
    pad_e = EP - N_EDGES
    rowp = jnp.concatenate([ei[0], jnp.zeros((pad_e,), jnp.int32)])
    colp = jnp.concatenate(
        [ei[1], jnp.full((pad_e,), N_NODES, jnp.int32)]
    )
    rowp2 = rowp.reshape(EP // CH, CH)
    colp2 = colp.reshape(EP // CH, CH)
    xP = jnp.concatenate(
        [x, jnp.zeros((NP - N_NODES, D), jnp.float32)]
    )
    batchP = jnp.concatenate(
        [batch.astype(jnp.int32),
         jnp.full((NP - N_NODES,), NUM_GRAPHS, jnp.int32)]
    ).reshape(NB, 1, BLK)
    b1r = b1.reshape(1, D)
    b2r = b2.reshape(1, D)
    bc1r = bc1.reshape(1, D)
    bc2r = bc2.reshape(1, N_CLASSES)

    degparts = _sc_deg(colp)
    y1 = _tc_y1(xP, W1, degparts)
    p1 = _sc_scatter(y1, rowp2, colp2)
    y2 = _tc_d1(p1, y1, degparts, b1r, W2)
    p2 = _sc_scatter(y2, rowp2, colp2)
    return _tc_d2(p2, y2, degparts, b2r, batchP, Wc1, bc1r, Wc2, bc2r)


# speedup vs baseline: 2.6971x; 1.0081x over previous
import functools

import jax
import jax.numpy as jnp
from jax import lax
from jax.experimental import pallas as pl
from jax.experimental.pallas import tpu as pltpu
from jax.experimental.pallas import tpu_sc as plsc

N_NODES = 10000
N_EDGES = 320000
D = 128
DH = D // 2
NUM_GRAPHS = 64
N_CLASSES = 16

NP = 10240
NB = 10
BLK = 1024
EP = 327680
CH = 128
NCHT = EP // (16 * CH)
NH = 4
NCHH = NCHT // NH
RING = 4
ROWS_PER_TILE = NP // 16


def _sc_mesh():
    return plsc.VectorSubcoreMesh(core_axis_name="c", subcore_axis_name="s")


def _sc_deg(colp):
    EPW = EP // 32

    @functools.partial(
        pl.kernel,
        out_type=jax.ShapeDtypeStruct((32, NP), jnp.float32),
        mesh=_sc_mesh(),
        scratch_types=[
            pltpu.VMEM((EPW,), jnp.int32),
            pltpu.VMEM((NP,), jnp.float32),
        ],
        compiler_params=pltpu.CompilerParams(needs_layout_passes=False),
    )
    def kdeg(col_hbm, out_hbm, colv, degv):
        cid = lax.axis_index("c")
        sid = lax.axis_index("s")
        wid = sid * 2 + cid
        pltpu.sync_copy(col_hbm.at[pl.ds(wid * EPW, EPW)], colv)

        def zero(j, carry):
            degv[pl.ds(j * 16, 16)] = jnp.zeros((16,), jnp.float32)
            return carry

        lax.fori_loop(0, NP // 16, zero, 0)

        ones = jnp.ones((16,), jnp.float32)

        def scat(i, carry):
            idx = colv[pl.ds(i * 16, 16)]
            plsc.addupdate_scatter(degv, [idx], ones)
            return carry

        lax.fori_loop(0, EPW // 16, scat, 0)
        pltpu.sync_copy(degv, out_hbm.at[wid])

    return kdeg(colp)


def _sc_scatter(ys, rowp2, colp2):
    @functools.partial(
        pl.kernel,
        out_type=jax.ShapeDtypeStruct((2, NP, DH), jnp.float32),
        mesh=_sc_mesh(),
        scratch_types=[
            pltpu.VMEM((NCHH, CH), jnp.int32),
            pltpu.VMEM((NCHH, CH), jnp.int32),
            pltpu.VMEM((RING, CH, DH), jnp.float32),
            pltpu.VMEM_SHARED((NP, DH), jnp.float32),
            pltpu.VMEM_SHARED((NP, DH), jnp.float32),
        ]
        + [pltpu.SemaphoreType.DMA] * (2 * RING),
        compiler_params=pltpu.CompilerParams(use_tc_tiling_on_sc=False),
    )
    def kscat(y_hbm, row_hbm, col_hbm, out_hbm, rowv, colv, datav, acc,
              ymem, *sems):
        gsems = sems[:RING]
        ssems = sems[RING:]
        cid = lax.axis_index("c")
        sid = lax.axis_index("s")
        ysrc = y_hbm.at[cid]

        tslice = pl.ds(sid * ROWS_PER_TILE, ROWS_PER_TILE)
        pltpu.sync_copy(ysrc.at[tslice], acc.at[tslice])
        pltpu.sync_copy(ysrc.at[tslice], ymem.at[tslice])
        plsc.subcore_barrier()

        def fire_g(g, b):
            pltpu.async_copy(ymem.at[rowv.at[g]], datav.at[b], gsems[b])

        def wait_g(g, b):
            pltpu.make_async_copy(
                ymem.at[rowv.at[g]], datav.at[b], gsems[b]
            ).wait()

        def fire_s(g, b):
            pltpu.async_copy(
                datav.at[b], acc.at[colv.at[g]], ssems[b], add=True
            )

        def wait_s(g, b):
            pltpu.make_async_copy(
                datav.at[b], acc.at[colv.at[g]], ssems[b]
            ).wait()

        for h in range(NH):
            pltpu.sync_copy(
                row_hbm.at[pl.ds(sid * NCHT + h * NCHH, NCHH)], rowv
            )
            pltpu.sync_copy(
                col_hbm.at[pl.ds(sid * NCHT + h * NCHH, NCHH)], colv
            )
            fire_g(0, 0)
            fire_g(1, 1)
            for g0 in range(2):
                wait_g(g0, g0)
                fire_s(g0, g0)
                fire_g(g0 + 2, g0 + 2)

            def outer(o, carry):
                for bi in range(RING):
                    g = 2 + o * RING + bi
                    b = (2 + bi) % RING
                    wait_g(g, b)
                    fire_s(g, b)
                    wait_s(g - 2, bi)
                    fire_g(g + 2, bi)
                return carry

            lax.fori_loop(0, (NCHH - 4) // RING, outer, 0)
            for g0 in range(NCHH - 2, NCHH):
                b = g0 % RING
                wait_g(g0, b)
                fire_s(g0, b)
                wait_s(g0 - 2, (g0 + 2) % RING)
            for g0 in range(NCHH - 2, NCHH):
                wait_s(g0, g0 % RING)

        plsc.subcore_barrier()
        pltpu.sync_copy(
            acc.at[pl.ds(sid * ROWS_PER_TILE, ROWS_PER_TILE)],
            out_hbm.at[cid, pl.ds(sid * ROWS_PER_TILE, ROWS_PER_TILE)],
        )

    return kscat(ys, rowp2, colp2)


def _dinv_of(deg_ref):
    dsum = jnp.sum(deg_ref[...], axis=0)
    return lax.rsqrt(1.0 + dsum)[:, None]


def _split(ref):
    return jnp.concatenate([ref[0], ref[1]], axis=1)


def _store_split(ref, val):
    ref[0] = val[:, :DH]
    ref[1] = val[:, DH:]


def _y1_body(x_ref, w_ref, deg_ref, y_ref):
    dinv = _dinv_of(deg_ref)
    xw = jnp.dot(x_ref[...], w_ref[...], preferred_element_type=jnp.float32)
    _store_split(y_ref, xw * dinv)


def _tc_y1(xP, W1, degparts):
    return pl.pallas_call(
        _y1_body,
        grid=(NB,),
        in_specs=[
            pl.BlockSpec((BLK, D), lambda i: (i, 0)),
            pl.BlockSpec((D, D), lambda i: (0, 0)),
            pl.BlockSpec((32, BLK), lambda i: (0, i)),
        ],
        out_specs=pl.BlockSpec((2, BLK, DH), lambda i: (0, i, 0)),
        out_shape=jax.ShapeDtypeStruct((2, NP, DH), jnp.float32),
    )(xP, W1, degparts)


def _d1_body(p_ref, deg_ref, b_ref, w_ref, out_ref):
    dinv = _dinv_of(deg_ref)
    h = jnp.maximum(dinv * _split(p_ref) + b_ref[...], 0.0)
    xw = jnp.dot(h, w_ref[...], preferred_element_type=jnp.float32)
    _store_split(out_ref, xw * dinv)


def _tc_d1(parts, degparts, b1r, W2):
    return pl.pallas_call(
        _d1_body,
        grid=(NB,),
        in_specs=[
            pl.BlockSpec((2, BLK, DH), lambda i: (0, i, 0)),
            pl.BlockSpec((32, BLK), lambda i: (0, i)),
            pl.BlockSpec((1, D), lambda i: (0, 0)),
            pl.BlockSpec((D, D), lambda i: (0, 0)),
        ],
        out_specs=pl.BlockSpec((2, BLK, DH), lambda i: (0, i, 0)),
        out_shape=jax.ShapeDtypeStruct((2, NP, DH), jnp.float32),
    )(parts, degparts, b1r, W2)


def _d2_body(p_ref, deg_ref, b_ref, batch_ref, wc1_ref, bc1_ref,
             wc2_ref, bc2_ref, out_ref, psum, cnt):
    i = pl.program_id(0)

    @pl.when(i == 0)
    def _():
        psum[...] = jnp.zeros_like(psum)
        cnt[...] = jnp.zeros_like(cnt)

    dinv = _dinv_of(deg_ref)
    h = jnp.maximum(dinv * _split(p_ref) + b_ref[...], 0.0)
    b = batch_ref[0, 0, :]
    oh = (b[:, None] == lax.broadcasted_iota(jnp.int32, (BLK, NUM_GRAPHS), 1))
    oh = oh.astype(jnp.float32)
    psum[...] += lax.dot_general(
        oh, h, (((0,), (0,)), ((), ())), preferred_element_type=jnp.float32
    )
    cnt[...] += jnp.broadcast_to(
        jnp.sum(oh, axis=0)[:, None], (NUM_GRAPHS, D)
    )

    @pl.when(i == NB - 1)
    def _():
        pooled = psum[...] / jnp.maximum(cnt[...], 1.0)
        z = jnp.maximum(
            jnp.dot(pooled, wc1_ref[...], preferred_element_type=jnp.float32)
            + bc1_ref[...],
            0.0,
        )
        out_ref[...] = (
            jnp.dot(z, wc2_ref[...], preferred_element_type=jnp.float32)
            + bc2_ref[...]
        )


def _tc_d2(parts, degparts, b2r, batchP, Wc1, bc1r, Wc2, bc2r):
    return pl.pallas_call(
        _d2_body,
        grid=(NB,),
        in_specs=[
            pl.BlockSpec((2, BLK, DH), lambda i: (0, i, 0)),
            pl.BlockSpec((32, BLK), lambda i: (0, i)),
            pl.BlockSpec((1, D), lambda i: (0, 0)),
            pl.BlockSpec((1, 1, BLK), lambda i: (i, 0, 0)),
            pl.BlockSpec((D, D), lambda i: (0, 0)),
            pl.BlockSpec((1, D), lambda i: (0, 0)),
            pl.BlockSpec((D, N_CLASSES), lambda i: (0, 0)),
            pl.BlockSpec((1, N_CLASSES), lambda i: (0, 0)),
        ],
        out_specs=pl.BlockSpec((NUM_GRAPHS, N_CLASSES), lambda i: (0, 0)),
        out_shape=jax.ShapeDtypeStruct((NUM_GRAPHS, N_CLASSES), jnp.float32),
        scratch_shapes=[
            pltpu.VMEM((NUM_GRAPHS, D), jnp.float32),
            pltpu.VMEM((NUM_GRAPHS, D), jnp.float32),
        ],
    )(parts, degparts, b2r, batchP, Wc1, bc1r, Wc2, bc2r)


def kernel(x, edge_index, batch, W1, b1, W2, b2, Wc1, bc1, Wc2, bc2):
    ei = edge_index.astype(jnp.int32)
    pad_e = EP - N_EDGES
    rowp = jnp.concatenate([ei[0], jnp.zeros((pad_e,), jnp.int32)])
    colp = jnp.concatenate(
        [ei[1], jnp.full((pad_e,), N_NODES, jnp.int32)]
    )
    rowp2 = rowp.reshape(EP // CH, CH)
    colp2 = colp.reshape(EP // CH, CH)
    xP = jnp.concatenate(
        [x, jnp.zeros((NP - N_NODES, D), jnp.float32)]
    )
    batchP = jnp.concatenate(
        [batch.astype(jnp.int32),
         jnp.full((NP - N_NODES,), NUM_GRAPHS, jnp.int32)]
    ).reshape(NB, 1, BLK)
    b1r = b1.reshape(1, D)
    b2r = b2.reshape(1, D)
    bc1r = bc1.reshape(1, D)
    bc2r = bc2.reshape(1, N_CLASSES)

    degparts = _sc_deg(colp)
    y1 = _tc_y1(xP, W1, degparts)
    p1 = _sc_scatter(y1, rowp2, colp2)
    y2 = _tc_d1(p1, degparts, b1r, W2)
    p2 = _sc_scatter(y2, rowp2, colp2)
    return _tc_d2(p2, degparts, b2r, batchP, Wc1, bc1r, Wc2, bc2r)
